# baseline (speedup 1.0000x reference)
import dataclasses

import jax
import jax.numpy as jnp
from jax import lax
from jax.experimental import pallas as pl
from jax.experimental.pallas import tpu as pltpu
from jax.experimental.pallas import tpu_sc as plsc

_CP = pltpu.CompilerParams()
if "needs_layout_passes" in pltpu.CompilerParams.__dataclass_fields__:
    _CP = dataclasses.replace(_CP, needs_layout_passes=False)
if "use_tc_tiling_on_sc" in pltpu.CompilerParams.__dataclass_fields__:
    _CP = dataclasses.replace(_CP, use_tc_tiling_on_sc=False)

_B = 16
_MAX_LEN = 4096
_D = 512
_TOTAL = 32768

_P = 64
_NW = 32
_NPIECES = (_B * _MAX_LEN) // _P
_PER_W = _NPIECES // _NW
_RD_SIZES = (64, 32, 16, 8, 4, 2, 1)


def _pad_and_stack_sc(flat, cu16):
    mesh = plsc.VectorSubcoreMesh(core_axis_name="c", subcore_axis_name="s")

    @pl.kernel(
        out_type=jax.ShapeDtypeStruct((_B * _MAX_LEN, _D), jnp.float32),
        mesh=mesh,
        compiler_params=_CP,
        scratch_types=[
            pltpu.VMEM((_P, _D), jnp.float32),
            pltpu.VMEM((_P, _D), jnp.float32),
            pltpu.VMEM((_P, _D), jnp.float32),
            pltpu.VMEM((16,), jnp.int32),
            pltpu.SemaphoreType.DMA,
            pltpu.SemaphoreType.DMA,
            pltpu.SemaphoreType.DMA,
        ],
    )
    def k(flat_hbm, cu_hbm, out_hbm, buf0, buf1, zbuf, cu_v,
          in_sem, out_sem0, out_sem1):
        bufs = (buf0, buf1)
        out_sems = (out_sem0, out_sem1)

        @pl.loop(0, _P)
        def _zero_row(r):
            for j in range(_D // 16):
                zbuf[r, pl.ds(j * 16, 16)] = jnp.zeros((16,), jnp.float32)

        pltpu.sync_copy(cu_hbm.at[pl.ds(0, 16)], cu_v)
        cuvec = cu_v[...]
        lane = lax.iota(jnp.int32, 16)

        def cu_at(i):
            v = jnp.sum(jnp.where(lane == i, cuvec, 0))
            return jnp.where(i >= _B, _TOTAL, v)

        def do_piece(i, par):
            slot = bufs[par]
            pidx = i * _NW + wid
            row0 = pidx * _P
            b = row0 // _MAX_LEN
            m0 = row0 % _MAX_LEN
            cu_b = cu_at(b)
            cu_b1 = cu_at(b + 1)
            nv = jnp.clip(cu_b1 - cu_b - m0, 0, _P)
            src = cu_b + m0

            @pl.when(i >= 2)
            def _drain_slot():
                pltpu.make_async_copy(
                    zbuf, out_hbm.at[pl.ds(row0, _P)], out_sems[par]).wait()

            @pl.when(nv == 0)
            def _all_pad():
                pltpu.make_async_copy(
                    zbuf, out_hbm.at[pl.ds(row0, _P)], out_sems[par]).start()

            @pl.when(nv > 0)
            def _data():
                off = jnp.int32(0)
                bits = []
                for sz in _RD_SIZES:
                    bit = (nv & sz) != 0
                    o = off

                    @pl.when(bit)
                    def _rd():
                        pltpu.make_async_copy(
                            flat_hbm.at[pl.ds(src + o, sz)],
                            slot.at[pl.ds(o, sz)], in_sem).start()

                    bits.append((bit, sz, o))
                    off = off + jnp.where(bit, sz, 0)
                for bit, sz, o in bits:
                    @pl.when(bit)
                    def _rd_wait():
                        pltpu.make_async_copy(
                            flat_hbm.at[pl.ds(src + o, sz)],
                            slot.at[pl.ds(o, sz)], in_sem).wait()

                @pl.loop(nv, _P)
                def _zero_tail(r):
                    for j in range(_D // 16):
                        slot[r, pl.ds(j * 16, 16)] = jnp.zeros(
                            (16,), jnp.float32)

                pltpu.make_async_copy(
                    slot, out_hbm.at[pl.ds(row0, _P)], out_sems[par]).start()

        wid = lax.axis_index("s") * 2 + lax.axis_index("c")

        @pl.loop(0, _PER_W, step=2)
        def _piece(i):
            do_piece(i, 0)
            do_piece(i + 1, 1)

        for par in range(2):
            pltpu.make_async_copy(
                zbuf, out_hbm.at[pl.ds(0, _P)], out_sems[par]).wait()

    return k(flat, cu16)


@jax.jit
def kernel(flat, cu_seqlens):
    cu16 = cu_seqlens[:16]
    out = _pad_and_stack_sc(flat, cu16)
    return out.reshape(_B, _MAX_LEN, _D)


# speedup vs baseline: 3.0421x; 3.0421x over previous
import dataclasses

import jax
import jax.numpy as jnp
from jax import lax
from jax.experimental import pallas as pl
from jax.experimental.pallas import tpu as pltpu
from jax.experimental.pallas import tpu_sc as plsc

_CP = pltpu.CompilerParams()
if "needs_layout_passes" in pltpu.CompilerParams.__dataclass_fields__:
    _CP = dataclasses.replace(_CP, needs_layout_passes=False)

_B = 16
_MAX_LEN = 4096
_D = 512
_TOTAL = 32768

_P = 64
_ZROWS = 32
_NW = 32
_NPIECES = (_B * _MAX_LEN) // _P
_PER_W = _NPIECES // _NW
_NSLOT = 3


def _pad_and_stack_sc(flat, cu16):
    mesh = plsc.VectorSubcoreMesh(core_axis_name="c", subcore_axis_name="s")

    @pl.kernel(
        out_type=jax.ShapeDtypeStruct((_B * _MAX_LEN, _D), jnp.float32),
        mesh=mesh,
        compiler_params=_CP,
        scratch_types=(
            [pltpu.VMEM((_P, _D), jnp.float32)] * _NSLOT
            + [pltpu.VMEM((_ZROWS, _D), jnp.float32)]
            + [pltpu.VMEM((_NSLOT, _P), jnp.int32)]
            + [pltpu.VMEM((16,), jnp.int32)]
            + [pltpu.SemaphoreType.DMA] * _NSLOT
            + [pltpu.SemaphoreType.DMA] * _NSLOT
        ),
    )
    def k(flat_hbm, cu_hbm, out_hbm, buf0, buf1, buf2, zbuf, idx_v, cu_v,
          isem0, isem1, isem2, osem0, osem1, osem2):
        bufs = (buf0, buf1, buf2)
        in_sems = (isem0, isem1, isem2)
        out_sems = (osem0, osem1, osem2)

        wid = lax.axis_index("s") * 2 + lax.axis_index("c")

        @pl.loop(0, _ZROWS)
        def _zero_row(r):
            for j in range(_D // 16):
                zbuf[r, pl.ds(j * 16, 16)] = jnp.zeros((16,), jnp.float32)

        pltpu.sync_copy(cu_hbm.at[pl.ds(0, 16)], cu_v)
        cuvec = cu_v[...]
        lane = lax.iota(jnp.int32, 16)

        def cu_at(i):
            v = jnp.sum(jnp.where(lane == i, cuvec, 0))
            return jnp.where(i >= _B, _TOTAL, v)

        def params(i):
            pidx = i * _NW + wid
            row0 = pidx * _P
            b = row0 // _MAX_LEN
            m0 = row0 % _MAX_LEN
            cu_b = cu_at(b)
            cu_b1 = cu_at(b + 1)
            nv = jnp.clip(cu_b1 - cu_b - m0, 0, _P)
            src = cu_b + m0
            return row0, nv, src

        def issue_read(i, s):
            row0, nv, src = params(i)

            @pl.when(nv > 0)
            def _():
                for q in range(_P // 16):
                    idx_v[s, pl.ds(q * 16, 16)] = jnp.minimum(
                        src + lane + (q * 16), _TOTAL - 1)
                pltpu.make_async_copy(
                    flat_hbm.at[idx_v.at[s]], bufs[s], in_sems[s]).start()

        def finish_piece(i, s):
            row0, nv, src = params(i)

            @pl.when(nv > 0)
            def _data():
                pltpu.make_async_copy(
                    flat_hbm.at[idx_v.at[s]], bufs[s], in_sems[s]).wait()

                @pl.loop(nv, _P)
                def _zero_tail(r):
                    for j in range(_D // 16):
                        bufs[s][r, pl.ds(j * 16, 16)] = jnp.zeros(
                            (16,), jnp.float32)

                pltpu.make_async_copy(
                    bufs[s], out_hbm.at[pl.ds(row0, _P)], out_sems[s]).start()

            @pl.when(nv == 0)
            def _all_pad():
                for h in range(2):
                    pltpu.make_async_copy(
                        zbuf, out_hbm.at[pl.ds(row0 + h * _ZROWS, _ZROWS)],
                        out_sems[s]).start()

        def drain_write(s):
            for _ in range(2):
                pltpu.make_async_copy(
                    zbuf, out_hbm.at[pl.ds(0, _ZROWS)], out_sems[s]).wait()

        def body(i, s, next_slot):
            finish_piece(i, s)
            if next_slot is not None:
                @pl.when(i >= 1)
                def _():
                    drain_write(next_slot)

                issue_read(i + 2, next_slot)

        issue_read(0, 0)
        issue_read(1, 1)

        @pl.loop(0, _PER_W - 2, step=_NSLOT)
        def _steady(i):
            for d in range(_NSLOT):
                body(i + d, d, (d + 2) % _NSLOT)

        body(_PER_W - 2, (_PER_W - 2) % _NSLOT, None)
        body(_PER_W - 1, (_PER_W - 1) % _NSLOT, None)
        for s in range(_NSLOT):
            drain_write(s)

    return k(flat, cu16)


@jax.jit
def kernel(flat, cu_seqlens):
    cu16 = cu_seqlens[:16]
    out = _pad_and_stack_sc(flat, cu16)
    return out.reshape(_B, _MAX_LEN, _D)


# speedup vs baseline: 3.0455x; 1.0011x over previous
import dataclasses

import jax
import jax.numpy as jnp
from jax import lax
from jax.experimental import pallas as pl
from jax.experimental.pallas import tpu as pltpu
from jax.experimental.pallas import tpu_sc as plsc

_CP = pltpu.CompilerParams()
if "needs_layout_passes" in pltpu.CompilerParams.__dataclass_fields__:
    _CP = dataclasses.replace(_CP, needs_layout_passes=False)

_B = 16
_MAX_LEN = 4096
_D = 512
_TOTAL = 32768

_P = 32
_NW = 32
_NPIECES = (_B * _MAX_LEN) // _P
_PER_W = _NPIECES // _NW
_NSLOT = 6
_LOOKAHEAD = 3


def _pad_and_stack_sc(flat, cu16):
    mesh = plsc.VectorSubcoreMesh(core_axis_name="c", subcore_axis_name="s")

    @pl.kernel(
        out_type=jax.ShapeDtypeStruct((_B * _MAX_LEN, _D), jnp.float32),
        mesh=mesh,
        compiler_params=_CP,
        scratch_types=(
            [pltpu.VMEM((_P, _D), jnp.float32)] * _NSLOT
            + [pltpu.VMEM((_P, _D), jnp.float32)]
            + [pltpu.VMEM((_NSLOT, _P), jnp.int32)]
            + [pltpu.VMEM((16,), jnp.int32)]
            + [pltpu.SemaphoreType.DMA] * _NSLOT
            + [pltpu.SemaphoreType.DMA] * _NSLOT
        ),
    )
    def k(flat_hbm, cu_hbm, out_hbm, *scratch):
        bufs = scratch[0:_NSLOT]
        zbuf = scratch[_NSLOT]
        idx_v = scratch[_NSLOT + 1]
        cu_v = scratch[_NSLOT + 2]
        in_sems = scratch[_NSLOT + 3:2 * _NSLOT + 3]
        out_sems = scratch[2 * _NSLOT + 3:3 * _NSLOT + 3]

        wid = lax.axis_index("s") * 2 + lax.axis_index("c")

        @pl.loop(0, _P)
        def _zero_row(r):
            for j in range(_D // 16):
                zbuf[r, pl.ds(j * 16, 16)] = jnp.zeros((16,), jnp.float32)

        pltpu.sync_copy(cu_hbm.at[pl.ds(0, 16)], cu_v)
        cuvec = cu_v[...]
        lane = lax.iota(jnp.int32, 16)

        def cu_at(i):
            v = jnp.sum(jnp.where(lane == i, cuvec, 0))
            return jnp.where(i >= _B, _TOTAL, v)

        def params(i):
            pidx = i * _NW + wid
            row0 = pidx * _P
            b = row0 // _MAX_LEN
            m0 = row0 % _MAX_LEN
            cu_b = cu_at(b)
            cu_b1 = cu_at(b + 1)
            nv = jnp.clip(cu_b1 - cu_b - m0, 0, _P)
            src = cu_b + m0
            return row0, nv, src

        def issue_read(i, s):
            row0, nv, src = params(i)

            @pl.when(nv > 0)
            def _():
                for q in range(_P // 16):
                    idx_v[s, pl.ds(q * 16, 16)] = jnp.minimum(
                        src + lane + (q * 16), _TOTAL - 1)
                pltpu.make_async_copy(
                    flat_hbm.at[idx_v.at[s]], bufs[s], in_sems[s]).start()

        def finish_piece(i, s):
            row0, nv, src = params(i)

            @pl.when(nv > 0)
            def _data():
                pltpu.make_async_copy(
                    flat_hbm.at[idx_v.at[s]], bufs[s], in_sems[s]).wait()

                @pl.loop(nv, _P)
                def _zero_tail(r):
                    for j in range(_D // 16):
                        bufs[s][r, pl.ds(j * 16, 16)] = jnp.zeros(
                            (16,), jnp.float32)

                pltpu.make_async_copy(
                    bufs[s], out_hbm.at[pl.ds(row0, _P)], out_sems[s]).start()

            @pl.when(nv == 0)
            def _all_pad():
                pltpu.make_async_copy(
                    zbuf, out_hbm.at[pl.ds(row0, _P)], out_sems[s]).start()

        def drain_write(s):
            pltpu.make_async_copy(
                zbuf, out_hbm.at[pl.ds(0, _P)], out_sems[s]).wait()

        def body(i, s, next_slot):
            finish_piece(i, s)
            if next_slot is not None:
                @pl.when(i >= _NSLOT - _LOOKAHEAD)
                def _():
                    drain_write(next_slot)

                issue_read(i + _LOOKAHEAD, next_slot)

        for p in range(_LOOKAHEAD):
            issue_read(p, p)

        _STEADY_END = ((_PER_W - _LOOKAHEAD) // _NSLOT) * _NSLOT

        @pl.loop(0, _STEADY_END, step=_NSLOT)
        def _steady(i):
            for d in range(_NSLOT):
                body(i + d, d, (d + _LOOKAHEAD) % _NSLOT)

        for p in range(_STEADY_END, _PER_W):
            s = p % _NSLOT
            nxt = p + _LOOKAHEAD
            body(p, s, nxt % _NSLOT if nxt < _PER_W else None)

        for p in range(_PER_W - _NSLOT, _PER_W):
            drain_write(p % _NSLOT)

    return k(flat, cu16)


@jax.jit
def kernel(flat, cu_seqlens):
    cu16 = cu_seqlens[:16]
    out = _pad_and_stack_sc(flat, cu16)
    return out.reshape(_B, _MAX_LEN, _D)
